# baseline (speedup 1.0000x reference)
import jax
import jax.numpy as jnp
from jax import lax
from jax.experimental import pallas as pl
from jax.experimental.pallas import tpu as pltpu
from jax.experimental.pallas import tpu_sc as plsc

B = 16384
D = 64
NUM_ENTITIES = 1000000
NUM_RELATIONS = 1000

_info = plsc.get_sparse_core_info()
NC, NS, L = _info.num_cores, _info.num_subcores, _info.num_lanes
NW = NC * NS
BPW = B // NW
C = 64
NCH = BPW // C
NPAIR = NCH // 2
NCOL = D // L
NG = C // 16

_GATHER_DNUMS = lax.GatherDimensionNumbers(
    offset_dims=(), collapsed_slice_dims=(0,), start_index_map=(0,))


def _shuffle(x, idx):
    return lax.gather(
        x, idx[:, None], _GATHER_DNUMS, slice_sizes=(1,),
        mode=lax.GatherScatterMode.PROMISE_IN_BOUNDS)


def _distmult_body(e_idc, p_idc, u_idc, node_p, edge_flat, out_hbm,
                   eidx, pidx, uidx, te, tu, e0, e1, u0, u1,
                   edge_v, out_v, s0, s1):
    wid = lax.axis_index("s") * NC + lax.axis_index("c")
    base = wid * BPW

    pltpu.sync_copy(e_idc.at[pl.ds(base, BPW)], eidx)
    pltpu.sync_copy(p_idc.at[pl.ds(base, BPW)], pidx)
    pltpu.sync_copy(u_idc.at[pl.ds(base, BPW)], uidx)
    pltpu.sync_copy(edge_flat, edge_v)

    def tix(i, carry):
        s = pl.ds(i * L, L)
        te[s] = eidx[s] >> 1
        tu[s] = uidx[s] >> 1
        return carry

    lax.fori_loop(0, BPW // L, tix, 0)

    def start(ch, ebuf, ubuf, sem):
        s = pl.ds(ch * C, C)
        pltpu.async_copy(node_p.at[te.at[s]], ebuf, sem)
        pltpu.async_copy(node_p.at[tu.at[s]], ubuf, sem)

    def drain(ebuf, ubuf, sem):
        dummy = node_p.at[te.at[pl.ds(0, C)]]
        pltpu.make_async_copy(dummy, ebuf, sem).wait()
        pltpu.make_async_copy(dummy, ubuf, sem).wait()

    lane = lax.iota(jnp.int32, L)

    def compute(ch, ebuf, ubuf):
        for g in range(NG):
            s = pl.ds(ch * C + g * 16, 16)
            pv = pidx[s] * D
            pe = (eidx[s] & 1) * D
            pu = (uidx[s] & 1) * D
            tot = jnp.zeros((L,), jnp.float32)
            for r in range(16):
                rr = g * 16 + r
                pb_r, pe_r, pu_r = pv[r], pe[r], pu[r]
                acc = None
                for c in range(NCOL):
                    t = (ebuf[rr, pl.ds(pe_r + c * L, L)]
                         * edge_v[pl.ds(pb_r + c * L, L)]
                         * ubuf[rr, pl.ds(pu_r + c * L, L)])
                    acc = t if acc is None else acc + t
                for sh in (8, 4, 2, 1):
                    acc = acc + _shuffle(acc, lane ^ sh)
                tot = jnp.where(lane == r, acc, tot)
            out_v[s] = tot

    start(0, e0, u0, s0)
    start(1, e1, u1, s1)

    def pair(k, carry):
        ch0 = 2 * k
        drain(e0, u0, s0)
        compute(ch0, e0, u0)

        @pl.when(k < NPAIR - 1)
        def _():
            start(ch0 + 2, e0, u0, s0)

        drain(e1, u1, s1)
        compute(ch0 + 1, e1, u1)

        @pl.when(k < NPAIR - 1)
        def _():
            start(ch0 + 3, e1, u1, s1)

        return carry

    lax.fori_loop(0, NPAIR, pair, 0)

    pltpu.sync_copy(out_v, out_hbm.at[pl.ds(base, BPW)])


_distmult = pl.kernel(
    _distmult_body,
    out_type=jax.ShapeDtypeStruct((B,), jnp.float32),
    mesh=plsc.VectorSubcoreMesh(core_axis_name="c", subcore_axis_name="s"),
    scratch_types=[
        pltpu.VMEM((BPW,), jnp.int32),
        pltpu.VMEM((BPW,), jnp.int32),
        pltpu.VMEM((BPW,), jnp.int32),
        pltpu.VMEM((BPW,), jnp.int32),
        pltpu.VMEM((BPW,), jnp.int32),
        pltpu.VMEM((C, 2 * D), jnp.float32),
        pltpu.VMEM((C, 2 * D), jnp.float32),
        pltpu.VMEM((C, 2 * D), jnp.float32),
        pltpu.VMEM((C, 2 * D), jnp.float32),
        pltpu.VMEM((NUM_RELATIONS * D,), jnp.float32),
        pltpu.VMEM((BPW,), jnp.float32),
        pltpu.SemaphoreType.DMA,
        pltpu.SemaphoreType.DMA,
    ],
)


def kernel(e_idc, p_idc, u_idc, feature_embeddings, node_embeddings,
           edge_embeddings):
    del feature_embeddings
    node_p = node_embeddings.reshape(NUM_ENTITIES // 2, 2 * D)
    edge_flat = edge_embeddings.reshape(NUM_RELATIONS * D)
    return _distmult(e_idc, p_idc, u_idc, node_p, edge_flat)


# speedup vs baseline: 1.6907x; 1.6907x over previous
import jax
import jax.numpy as jnp
from jax import lax
from jax.experimental import pallas as pl
from jax.experimental.pallas import tpu as pltpu
from jax.experimental.pallas import tpu_sc as plsc

B = 16384
D = 64
NUM_ENTITIES = 1000000
NUM_RELATIONS = 1000

_info = plsc.get_sparse_core_info()
NC, NS, L = _info.num_cores, _info.num_subcores, _info.num_lanes
NW = NC * NS
BPW = B // NW
G = 16
NGRP = BPW // G
NPAIR = NGRP // 2
NCOL = D // L

_GATHER_DNUMS = lax.GatherDimensionNumbers(
    offset_dims=(), collapsed_slice_dims=(0,), start_index_map=(0,))


def _shuffle(x, idx):
    return lax.gather(
        x, idx[:, None], _GATHER_DNUMS, slice_sizes=(1,),
        mode=lax.GatherScatterMode.PROMISE_IN_BOUNDS)


def _distmult_body(e_idc, p_idc, u_idc, node, edge_flat, out_hbm,
                   eidx, pidx, uidx, ebuf, ubuf, edge_v, out_v, se, su):
    wid = lax.axis_index("s") * NC + lax.axis_index("c")
    base = wid * BPW

    pltpu.sync_copy(e_idc.at[pl.ds(base, BPW)], eidx)
    pltpu.sync_copy(p_idc.at[pl.ds(base, BPW)], pidx)
    pltpu.sync_copy(u_idc.at[pl.ds(base, BPW)], uidx)
    pltpu.sync_copy(edge_flat, edge_v)

    lane = lax.iota(jnp.int32, L)

    def start(g, b):
        off = g * G
        ev = eidx[pl.ds(off, G)]
        uv = uidx[pl.ds(off, G)]
        for r in range(G):
            pltpu.async_copy(node.at[pl.ds(ev[r], 1)],
                             ebuf.at[pl.ds(b * G + r, 1)], se)
            pltpu.async_copy(node.at[pl.ds(uv[r], 1)],
                             ubuf.at[pl.ds(b * G + r, 1)], su)

    def drain(b):
        dummy = node.at[pl.ds(0, G)]
        pltpu.make_async_copy(dummy, ebuf.at[pl.ds(b * G, G)], se).wait()
        pltpu.make_async_copy(dummy, ubuf.at[pl.ds(b * G, G)], su).wait()

    def compute(g, b):
        s = pl.ds(g * G, G)
        pv = pidx[s] * D
        tot = jnp.zeros((L,), jnp.float32)
        for r in range(G):
            rr = b * G + r
            pb_r = pv[r]
            acc = None
            for c in range(NCOL):
                t = (ebuf[rr, pl.ds(c * L, L)]
                     * edge_v[pl.ds(pb_r + c * L, L)]
                     * ubuf[rr, pl.ds(c * L, L)])
                acc = t if acc is None else acc + t
            for sh in (8, 4, 2, 1):
                acc = acc + _shuffle(acc, lane ^ sh)
            tot = jnp.where(lane == r, acc, tot)
        out_v[s] = tot

    start(0, 0)
    start(1, 1)

    def pair(k, carry):
        g0 = 2 * k
        drain(0)
        compute(g0, 0)

        @pl.when(k < NPAIR - 1)
        def _():
            start(g0 + 2, 0)

        drain(1)
        compute(g0 + 1, 1)

        @pl.when(k < NPAIR - 1)
        def _():
            start(g0 + 3, 1)

        return carry

    lax.fori_loop(0, NPAIR, pair, 0)

    pltpu.sync_copy(out_v, out_hbm.at[pl.ds(base, BPW)])


_distmult = pl.kernel(
    _distmult_body,
    out_type=jax.ShapeDtypeStruct((B,), jnp.float32),
    mesh=plsc.VectorSubcoreMesh(core_axis_name="c", subcore_axis_name="s"),
    scratch_types=[
        pltpu.VMEM((BPW,), jnp.int32),
        pltpu.VMEM((BPW,), jnp.int32),
        pltpu.VMEM((BPW,), jnp.int32),
        pltpu.VMEM((2 * G, D), jnp.float32),
        pltpu.VMEM((2 * G, D), jnp.float32),
        pltpu.VMEM((NUM_RELATIONS * D,), jnp.float32),
        pltpu.VMEM((BPW,), jnp.float32),
        pltpu.SemaphoreType.DMA,
        pltpu.SemaphoreType.DMA,
    ],
)


def kernel(e_idc, p_idc, u_idc, feature_embeddings, node_embeddings,
           edge_embeddings):
    del feature_embeddings
    edge_flat = edge_embeddings.reshape(NUM_RELATIONS * D)
    return _distmult(e_idc, p_idc, u_idc, node_embeddings, edge_flat)
